# baseline (speedup 1.0000x reference)
import functools

import jax
import jax.numpy as jnp
from jax import lax
from jax.experimental import pallas as pl
from jax.experimental.pallas import tpu as pltpu
from jax.experimental.pallas import tpu_sc as plsc

NGBUF = 8
NTBUF = 2


@functools.partial(jax.jit, static_argnums=(2, 3, 4))
def _sc_gather(idx_t, table, B0, B1, D):
    info = plsc.get_sparse_core_info()
    NC, NS = info.num_cores, info.num_subcores
    NW = NC * NS
    LANES = 128
    DT = D // 8
    NB0 = B0 // LANES
    n_blocks = B1 * NB0 // NW
    mesh = plsc.VectorSubcoreMesh(core_axis_name="c", subcore_axis_name="s")

    @functools.partial(
        pl.kernel,
        out_type=jax.ShapeDtypeStruct((B1, DT, NB0, 8, LANES), jnp.float32),
        mesh=mesh,
        scratch_types=[
            pltpu.VMEM((n_blocks * LANES,), jnp.int32),
            pltpu.VMEM((NGBUF, LANES, D), jnp.float32),
            pltpu.VMEM((NTBUF, DT, 8, LANES), jnp.float32),
            pltpu.SemaphoreType.DMA,
            pltpu.SemaphoreType.DMA,
            pltpu.SemaphoreType.DMA,
            pltpu.SemaphoreType.DMA,
            pltpu.SemaphoreType.DMA,
            pltpu.SemaphoreType.DMA,
            pltpu.SemaphoreType.DMA,
            pltpu.SemaphoreType.DMA,
            pltpu.SemaphoreType.DMA,
            pltpu.SemaphoreType.DMA,
        ],
        compiler_params=pltpu.CompilerParams(
            use_tc_tiling_on_sc=False, needs_layout_passes=False
        ),
    )
    def k(idx_hbm, table_hbm, out_hbm, idx_v, gbuf, tbuf, g0, g1, g2, g3, g4, g5, g6, g7, w0, w1):
        wid = lax.axis_index("s") * NC + lax.axis_index("c")
        m0 = wid * n_blocks
        gsem = (g0, g1, g2, g3, g4, g5, g6, g7)
        wsem = (w0, w1)
        lane = lax.iota(jnp.int32, 16)
        row16 = [lane + 16 * q for q in range(8)]

        pltpu.sync_copy(idx_hbm.at[pl.ds(m0 * LANES, n_blocks * LANES)], idx_v)

        def issue_gather(mloc, b):
            pltpu.async_copy(
                table_hbm.at[idx_v.at[pl.ds(mloc * LANES, LANES)]],
                gbuf.at[b],
                gsem[b],
            )

        def out_slab(mloc):
            m = m0 + mloc
            return out_hbm.at[m // NB0, :, m % NB0]

        for b in range(NGBUF):
            issue_gather(b, b)

        def body(g, carry):
            for b in range(NGBUF):
                mloc = g * NGBUF + b
                tb = b % NTBUF

                @pl.when(mloc >= NTBUF)
                def _():
                    pltpu.make_async_copy(
                        tbuf.at[tb], out_slab(mloc - NTBUF), wsem[tb]
                    ).wait()

                pltpu.make_async_copy(
                    table_hbm.at[idx_v.at[pl.ds(mloc * LANES, LANES)]],
                    gbuf.at[b],
                    gsem[b],
                ).wait()

                @plsc.parallel_loop(0, D, unroll=2)
                def tr(d):
                    col = (d + lane) & (D - 1)
                    dtv = col >> 3
                    dsv = col & 7
                    for q in range(8):
                        v = plsc.load_gather(gbuf.at[b], [row16[q], col])
                        plsc.store_scatter(
                            tbuf.at[tb], [dtv, dsv, row16[q]], v
                        )

                pltpu.async_copy(tbuf.at[tb], out_slab(mloc), wsem[tb])

                @pl.when(mloc + NGBUF < n_blocks)
                def _():
                    issue_gather(mloc + NGBUF, b)
            return carry

        lax.fori_loop(0, n_blocks // NGBUF, body, 0)

        for tb in range(NTBUF):
            last = n_blocks - NTBUF + tb
            pltpu.make_async_copy(tbuf.at[tb], out_slab(last), wsem[tb]).wait()

    return k(idx_t, table)


def kernel(x, label_embedding_weight):
    B0, B1, _ = x.shape
    D = label_embedding_weight.shape[1]
    idx_t = x[:, :, 1].astype(jnp.int32).T.reshape(B0 * B1)
    out5 = _sc_gather(idx_t, label_embedding_weight, B0, B1, D)
    return out5.transpose(2, 4, 0, 1, 3).reshape(B0, B1, D)


# speedup vs baseline: 1.0079x; 1.0079x over previous
import functools

import jax
import jax.numpy as jnp
from jax import lax
from jax.experimental import pallas as pl
from jax.experimental.pallas import tpu as pltpu
from jax.experimental.pallas import tpu_sc as plsc

NGBUF = 4
NTBUF = 2


@functools.partial(jax.jit, static_argnums=(2, 3, 4))
def _sc_gather(idx_t, table, B0, B1, D):
    info = plsc.get_sparse_core_info()
    NC, NS = info.num_cores, info.num_subcores
    NW = NC * NS
    LANES = 128
    DT = D // 8
    NB0 = B0 // LANES
    n_blocks = B1 * NB0 // NW
    mesh = plsc.VectorSubcoreMesh(core_axis_name="c", subcore_axis_name="s")

    @functools.partial(
        pl.kernel,
        out_type=jax.ShapeDtypeStruct((B1, DT, NB0, 8, LANES), jnp.float32),
        mesh=mesh,
        scratch_types=[
            pltpu.VMEM((n_blocks * LANES,), jnp.int32),
            pltpu.VMEM((NGBUF, LANES, D), jnp.float32),
            pltpu.VMEM((NTBUF, DT, 8, LANES), jnp.float32),
            pltpu.SemaphoreType.DMA,
            pltpu.SemaphoreType.DMA,
            pltpu.SemaphoreType.DMA,
            pltpu.SemaphoreType.DMA,
            pltpu.SemaphoreType.DMA,
            pltpu.SemaphoreType.DMA,
        ],
        compiler_params=pltpu.CompilerParams(
            use_tc_tiling_on_sc=False, needs_layout_passes=False
        ),
    )
    def k(idx_hbm, table_hbm, out_hbm, idx_v, gbuf, tbuf, g0, g1, g2, g3, w0, w1):
        wid = lax.axis_index("s") * NC + lax.axis_index("c")
        m0 = wid * n_blocks
        gsem = (g0, g1, g2, g3)
        wsem = (w0, w1)
        lane = lax.iota(jnp.int32, 16)
        row16 = [lane + 16 * q for q in range(8)]

        pltpu.sync_copy(idx_hbm.at[pl.ds(m0 * LANES, n_blocks * LANES)], idx_v)

        def issue_gather(mloc, b):
            pltpu.async_copy(
                table_hbm.at[idx_v.at[pl.ds(mloc * LANES, LANES)]],
                gbuf.at[b],
                gsem[b],
            )

        def out_slab(mloc):
            m = m0 + mloc
            return out_hbm.at[m // NB0, :, m % NB0]

        for b in range(NGBUF):
            issue_gather(b, b)

        def body(g, carry):
            for b in range(NGBUF):
                mloc = g * NGBUF + b
                tb = b % NTBUF

                @pl.when(mloc >= NTBUF)
                def _():
                    pltpu.make_async_copy(
                        tbuf.at[tb], out_slab(mloc - NTBUF), wsem[tb]
                    ).wait()

                pltpu.make_async_copy(
                    table_hbm.at[idx_v.at[pl.ds(mloc * LANES, LANES)]],
                    gbuf.at[b],
                    gsem[b],
                ).wait()

                @plsc.parallel_loop(0, D, unroll=4)
                def tr(d):
                    col = (d + lane) & (D - 1)
                    dtv = col >> 3
                    dsv = col & 7
                    for q in range(8):
                        v = plsc.load_gather(gbuf.at[b], [row16[q], col])
                        plsc.store_scatter(
                            tbuf.at[tb], [dtv, dsv, row16[q]], v
                        )

                pltpu.async_copy(tbuf.at[tb], out_slab(mloc), wsem[tb])

                @pl.when(mloc + NGBUF < n_blocks)
                def _():
                    issue_gather(mloc + NGBUF, b)
            return carry

        lax.fori_loop(0, n_blocks // NGBUF, body, 0)

        for tb in range(NTBUF):
            last = n_blocks - NTBUF + tb
            pltpu.make_async_copy(tbuf.at[tb], out_slab(last), wsem[tb]).wait()

    return k(idx_t, table)


def kernel(x, label_embedding_weight):
    B0, B1, _ = x.shape
    D = label_embedding_weight.shape[1]
    idx_t = x[:, :, 1].astype(jnp.int32).T.reshape(B0 * B1)
    out5 = _sc_gather(idx_t, label_embedding_weight, B0, B1, D)
    return out5.transpose(2, 4, 0, 1, 3).reshape(B0, B1, D)
